# baseline (speedup 1.0000x reference)
import functools

import jax
import jax.numpy as jnp
from jax import lax
from jax.experimental import pallas as pl
from jax.experimental.pallas import tpu as pltpu
from jax.experimental.pallas import tpu_sc as plsc

N_FINE = 100000
C = 128
G = 128
NUM_G_FULL = N_FINE // G
TAIL = N_FINE - NUM_G_FULL * G
TAIL_BASE = NUM_G_FULL * G

_info = plsc.get_sparse_core_info()
NC, NS = _info.num_cores, _info.num_subcores
NW = NC * NS

K_HI = -(-NUM_G_FULL // NW)
K_LO = NUM_G_FULL // NW
N_BIG = NUM_G_FULL - K_LO * NW

NBUF = 4
D = 2
ROUNDS = -(-(K_HI + D) // NBUF)

L = 16

COORD_N = N_FINE * 3
CW = 25
COORD_PER_W = COORD_N // CW


def _make_gather():
    mesh = plsc.VectorSubcoreMesh(core_axis_name="c", subcore_axis_name="s")

    scratch = (
        [pltpu.VMEM((G,), jnp.int32) for _ in range(NBUF)]
        + [pltpu.VMEM((G, C), jnp.float32) for _ in range(NBUF)]
        + [pltpu.VMEM((COORD_PER_W,), jnp.float32),
           pltpu.VMEM((8,), jnp.int32)]
        + [pltpu.SemaphoreType.DMA for _ in range(2 * NBUF + 1)]
    )

    @functools.partial(
        pl.kernel,
        mesh=mesh,
        out_type=(
            jax.ShapeDtypeStruct((N_FINE, C), jnp.float32),
            jax.ShapeDtypeStruct((COORD_N,), jnp.float32),
            jax.ShapeDtypeStruct((4,), jnp.int32),
        ),
        scratch_types=scratch,
    )
    def gather_kernel(feat_hbm, cluster_hbm, coord_hbm, offset_hbm,
                      out_hbm, coord_out, offset_out, *bufs):
        idx = bufs[:NBUF]
        rows = bufs[NBUF:2 * NBUF]
        cbuf, obuf = bufs[2 * NBUF:2 * NBUF + 2]
        gsem = bufs[2 * NBUF + 2:3 * NBUF + 2]
        ssem = bufs[3 * NBUF + 2:4 * NBUF + 2]
        csem = bufs[4 * NBUF + 2]

        wid = lax.axis_index("s") * NC + lax.axis_index("c")
        is_big = wid < N_BIG
        n_chunks = jnp.where(is_big, K_HI, K_LO)
        base_chunk = K_LO * wid + jnp.minimum(wid, N_BIG)

        cbase = wid * COORD_PER_W

        @pl.when(wid < CW)
        def _():
            pltpu.make_async_copy(coord_hbm.at[pl.ds(cbase, COORD_PER_W)],
                                  cbuf, csem).start()

        def gather_of(k, b):
            return pltpu.make_async_copy(feat_hbm.at[idx[b]], rows[b], gsem[b])

        def store_of(k, b):
            return pltpu.make_async_copy(
                rows[b], out_hbm.at[pl.ds((base_chunk + k) * G, G)], ssem[b])

        def round_body(r, carry):
            k0 = r * NBUF
            for b in range(NBUF):
                k = k0 + b

                bc = (b - D) % NBUF
                kc = k - D

                @pl.when((kc >= 0) & (kc < n_chunks))
                def _(kc=kc, bc=bc):
                    gather_of(kc, bc).wait()
                    store_of(kc, bc).start()

                @pl.when(k < n_chunks)
                def _(k=k, b=b):
                    @pl.when(k >= NBUF)
                    def _():
                        store_of(k - NBUF, b).wait()

                    pltpu.sync_copy(
                        cluster_hbm.at[pl.ds((base_chunk + k) * G, G)], idx[b])
                    gather_of(k, b).start()

            return carry

        lax.fori_loop(0, ROUNDS, round_body, 0)

        for b in range(NBUF):
            last_k = (n_chunks - 1) - ((n_chunks - 1 - b) % NBUF)

            @pl.when(last_k >= 0)
            def _(last_k=last_k, b=b):
                store_of(last_k, b).wait()

        @pl.when(wid < CW)
        def _():
            pltpu.make_async_copy(coord_hbm.at[pl.ds(cbase, COORD_PER_W)],
                                  cbuf, csem).wait()
            pltpu.sync_copy(cbuf, coord_out.at[pl.ds(cbase, COORD_PER_W)])

        @pl.when(wid == NW - 2)
        def _():
            pltpu.sync_copy(offset_hbm, obuf.at[pl.ds(0, 4)])
            pltpu.sync_copy(obuf.at[pl.ds(0, 4)], offset_out)

        @pl.when(wid == NW - 1)
        def _():
            for j in range(G // L):
                idx[0][pl.ds(j * L, L)] = jnp.zeros((L,), jnp.int32)
            pltpu.sync_copy(cluster_hbm.at[pl.ds(TAIL_BASE, TAIL)],
                            idx[0].at[pl.ds(0, TAIL)])
            pltpu.async_copy(feat_hbm.at[idx[0]], rows[0], gsem[0]).wait()
            pltpu.sync_copy(rows[0].at[pl.ds(0, TAIL)],
                            out_hbm.at[pl.ds(TAIL_BASE, TAIL)])

    return gather_kernel


_gather = _make_gather()


def kernel(coord, feat, offset, cluster):
    feat_out, coord_flat, offset_out = _gather(
        feat, cluster, coord.reshape(-1), offset)
    return (coord_flat.reshape(N_FINE, 3), feat_out, offset_out)


# speedup vs baseline: 2.8981x; 2.8981x over previous
import functools

import jax
import jax.numpy as jnp
from jax import lax
from jax.experimental import pallas as pl
from jax.experimental.pallas import tpu as pltpu
from jax.experimental.pallas import tpu_sc as plsc

N_FINE = 100000
C = 128
G = 128
NUM_G_FULL = N_FINE // G
TAIL = N_FINE - NUM_G_FULL * G
TAIL_BASE = NUM_G_FULL * G

_info = plsc.get_sparse_core_info()
NC, NS = _info.num_cores, _info.num_subcores
NW = NC * NS

K_HI = -(-NUM_G_FULL // NW)
K_LO = NUM_G_FULL // NW
N_BIG = NUM_G_FULL - K_LO * NW

NBUF = 3
NS_HI = -(-K_HI // 2)
ROUNDS = -(-(NS_HI + 1) // NBUF)

L = 16


def _make_gather():
    mesh = plsc.VectorSubcoreMesh(core_axis_name="c", subcore_axis_name="s")

    scratch = (
        [pltpu.VMEM((G,), jnp.int32) for _ in range(2 * NBUF)]
        + [pltpu.VMEM((2 * G, C), jnp.float32) for _ in range(NBUF)]
        + [pltpu.SemaphoreType.DMA for _ in range(2 * NBUF)]
    )

    @functools.partial(
        pl.kernel,
        mesh=mesh,
        out_type=jax.ShapeDtypeStruct((N_FINE, C), jnp.float32),
        scratch_types=scratch,
    )
    def gather_kernel(feat_hbm, cluster_hbm, out_hbm, *bufs):
        idx = bufs[:2 * NBUF]
        rows = bufs[2 * NBUF:3 * NBUF]
        gsem = bufs[3 * NBUF:4 * NBUF]
        ssem = bufs[4 * NBUF:]

        wid = lax.axis_index("s") * NC + lax.axis_index("c")
        is_big = wid < N_BIG
        n_chunks = jnp.where(is_big, K_HI, K_LO)
        n_supers = (n_chunks + 1) // 2
        base_chunk = K_LO * wid + jnp.minimum(wid, N_BIG)

        def gather_of(half, b):
            return pltpu.make_async_copy(
                feat_hbm.at[idx[2 * b + half]],
                rows[b].at[pl.ds(half * G, G)], gsem[b])

        def store_full(s, b):
            return pltpu.make_async_copy(
                rows[b],
                out_hbm.at[pl.ds((base_chunk + 2 * s) * G, 2 * G)], ssem[b])

        def store_half(s, b):
            return pltpu.make_async_copy(
                rows[b].at[pl.ds(0, G)],
                out_hbm.at[pl.ds((base_chunk + 2 * s) * G, G)], ssem[b])

        def launch_one(s, b, half):
            k = 2 * s + half
            pltpu.sync_copy(cluster_hbm.at[pl.ds((base_chunk + k) * G, G)],
                            idx[2 * b + half])
            gather_of(half, b).start()

        def round_body(r, carry):
            s0 = r * NBUF
            for b in range(NBUF):
                s = s0 + b

                bc = (b - 1) % NBUF
                sc = s - 1

                @pl.when((sc >= 0) & (2 * sc + 1 < n_chunks))
                def _(sc=sc, bc=bc):
                    gather_of(0, bc).wait()
                    gather_of(1, bc).wait()
                    store_full(sc, bc).start()

                @pl.when((sc >= 0) & (2 * sc + 1 == n_chunks))
                def _(sc=sc, bc=bc):
                    gather_of(0, bc).wait()
                    store_half(sc, bc).start()

                @pl.when(2 * s < n_chunks)
                def _(s=s, b=b):
                    @pl.when(s >= NBUF)
                    def _(s=s, b=b):
                        store_full(s - NBUF, b).wait()

                    launch_one(s, b, 0)

                    @pl.when(2 * s + 1 < n_chunks)
                    def _(s=s, b=b):
                        launch_one(s, b, 1)

            return carry

        lax.fori_loop(0, ROUNDS, round_body, 0)

        for b in range(NBUF):
            last_s = (n_supers - 1) - ((n_supers - 1 - b) % NBUF)

            @pl.when((last_s >= 0) & (2 * last_s + 1 < n_chunks))
            def _(last_s=last_s, b=b):
                store_full(last_s, b).wait()

            @pl.when((last_s >= 0) & (2 * last_s + 1 == n_chunks))
            def _(last_s=last_s, b=b):
                store_half(last_s, b).wait()

        @pl.when(wid == NW - 1)
        def _():
            for j in range(G // L):
                idx[0][pl.ds(j * L, L)] = jnp.zeros((L,), jnp.int32)
            pltpu.sync_copy(cluster_hbm.at[pl.ds(TAIL_BASE, TAIL)],
                            idx[0].at[pl.ds(0, TAIL)])
            pltpu.async_copy(feat_hbm.at[idx[0]],
                             rows[0].at[pl.ds(0, G)], gsem[0]).wait()
            pltpu.sync_copy(rows[0].at[pl.ds(0, TAIL)],
                            out_hbm.at[pl.ds(TAIL_BASE, TAIL)])

    return gather_kernel


_gather = _make_gather()


def kernel(coord, feat, offset, cluster):
    feat_out = _gather(feat, cluster)
    return (coord, feat_out, offset)


# speedup vs baseline: 3.1802x; 1.0973x over previous
import functools

import jax
import jax.numpy as jnp
from jax import lax
from jax.experimental import pallas as pl
from jax.experimental.pallas import tpu as pltpu
from jax.experimental.pallas import tpu_sc as plsc

N_FINE = 100000
C = 128
G = 128
NUM_G_FULL = N_FINE // G
TAIL = N_FINE - NUM_G_FULL * G
TAIL_BASE = NUM_G_FULL * G

_info = plsc.get_sparse_core_info()
NC, NS = _info.num_cores, _info.num_subcores
NW = NC * NS

K_HI = -(-NUM_G_FULL // NW)
K_LO = NUM_G_FULL // NW
N_BIG = NUM_G_FULL - K_LO * NW

NBUF = 7
D = 3
ROUNDS = -(-(K_HI + D) // NBUF)

L = 16


def _make_gather():
    mesh = plsc.VectorSubcoreMesh(core_axis_name="c", subcore_axis_name="s")

    scratch = (
        [pltpu.VMEM((G,), jnp.int32) for _ in range(NBUF)]
        + [pltpu.VMEM((G, C), jnp.float32) for _ in range(NBUF)]
        + [pltpu.SemaphoreType.DMA for _ in range(2 * NBUF)]
    )

    @functools.partial(
        pl.kernel,
        mesh=mesh,
        out_type=jax.ShapeDtypeStruct((N_FINE, C), jnp.float32),
        scratch_types=scratch,
    )
    def gather_kernel(feat_hbm, cluster_hbm, out_hbm, *bufs):
        idx = bufs[:NBUF]
        rows = bufs[NBUF:2 * NBUF]
        gsem = bufs[2 * NBUF:3 * NBUF]
        ssem = bufs[3 * NBUF:]

        wid = lax.axis_index("s") * NC + lax.axis_index("c")
        is_big = wid < N_BIG
        n_chunks = jnp.where(is_big, K_HI, K_LO)
        base_chunk = K_LO * wid + jnp.minimum(wid, N_BIG)

        def gather_of(k, b):
            return pltpu.make_async_copy(feat_hbm.at[idx[b]], rows[b], gsem[b])

        def store_of(k, b):
            return pltpu.make_async_copy(
                rows[b], out_hbm.at[pl.ds((base_chunk + k) * G, G)], ssem[b])

        def round_body(r, carry):
            k0 = r * NBUF
            for b in range(NBUF):
                k = k0 + b

                bc = (b - D) % NBUF
                kc = k - D

                @pl.when((kc >= 0) & (kc < n_chunks))
                def _(kc=kc, bc=bc):
                    gather_of(kc, bc).wait()
                    store_of(kc, bc).start()

                @pl.when(k < n_chunks)
                def _(k=k, b=b):
                    @pl.when(k >= NBUF)
                    def _():
                        store_of(k - NBUF, b).wait()

                    pltpu.sync_copy(
                        cluster_hbm.at[pl.ds((base_chunk + k) * G, G)], idx[b])
                    gather_of(k, b).start()

            return carry

        lax.fori_loop(0, ROUNDS, round_body, 0)

        for b in range(NBUF):
            last_k = (n_chunks - 1) - ((n_chunks - 1 - b) % NBUF)

            @pl.when(last_k >= 0)
            def _(last_k=last_k, b=b):
                store_of(last_k, b).wait()

        @pl.when(wid == NW - 1)
        def _():
            for j in range(G // L):
                idx[0][pl.ds(j * L, L)] = jnp.zeros((L,), jnp.int32)
            pltpu.sync_copy(cluster_hbm.at[pl.ds(TAIL_BASE, TAIL)],
                            idx[0].at[pl.ds(0, TAIL)])
            pltpu.async_copy(feat_hbm.at[idx[0]], rows[0], gsem[0]).wait()
            pltpu.sync_copy(rows[0].at[pl.ds(0, TAIL)],
                            out_hbm.at[pl.ds(TAIL_BASE, TAIL)])

    return gather_kernel


_gather = _make_gather()


def kernel(coord, feat, offset, cluster):
    feat_out = _gather(feat, cluster)
    return (coord, feat_out, offset)
